# baseline (speedup 1.0000x reference)
import functools

import jax
import jax.numpy as jnp
from jax import lax
from jax.experimental import pallas as pl
from jax.experimental.pallas import tpu as pltpu
from jax.experimental.pallas import tpu_sc as plsc

N = 10000
E = 320000
D = 128
C = 16

NC = 2
NS = 16
CHUNK = 128

E_TOT = E + N
NBUF = 2
NCHUNKS = 84
HALF = NCHUNKS // 2
EPW = NCHUNKS * CHUNK
E_PAD = EPW * NC * NS
R = 10112
RPW = R // NS

_mesh = plsc.VectorSubcoreMesh(core_axis_name="c", subcore_axis_name="s")


@jax.jit
def _sc_degree(dstp, ones_rows, zrows):
    @functools.partial(
        pl.kernel,
        mesh=_mesh,
        out_type=jax.ShapeDtypeStruct((NC, R, 16), jnp.float32),
        scratch_types=[
            pltpu.VMEM((2, HALF, CHUNK), jnp.int32),
            pltpu.VMEM((CHUNK, 16), jnp.float32),
            pltpu.VMEM_SHARED((R, 16), jnp.float32),
            pltpu.SemaphoreType.DMA,
        ],
    )
    def deg_kernel(dstp_hbm, ones_hbm, z_hbm, out_hbm, didx_v, ones_v, acc_sh,
                   sem):
        c = lax.axis_index("c")
        s = lax.axis_index("s")
        w = c * NS + s
        pltpu.sync_copy(dstp_hbm.at[w], didx_v)
        pltpu.sync_copy(ones_hbm, ones_v)
        pltpu.sync_copy(z_hbm, acc_sh.at[pl.ds(s * RPW, RPW)])
        plsc.subcore_barrier()

        GRP = 14
        for h in range(2):
            @pl.loop(0, HALF // GRP)
            def _(gi):
                @pl.loop(0, GRP)
                def _(j):
                    pltpu.async_copy(
                        ones_v, acc_sh.at[didx_v.at[h].at[gi * GRP + j]],
                        sem, add=True)

                @pl.loop(0, GRP)
                def _(j):
                    pltpu.make_async_copy(
                        ones_v, acc_sh.at[didx_v.at[0].at[0]], sem).wait()

        plsc.subcore_barrier()
        pltpu.sync_copy(acc_sh.at[pl.ds(s * RPW, RPW)],
                        out_hbm.at[c].at[pl.ds(s * RPW, RPW)])

    return deg_kernel(dstp, ones_rows, zrows)


@jax.jit
def _sc_propagate(xs, srcp, dstp, zrows):
    @functools.partial(
        pl.kernel,
        mesh=_mesh,
        out_type=jax.ShapeDtypeStruct((NC, R, D), jnp.float32),
        scratch_types=(
            [pltpu.VMEM((HALF + NBUF, CHUNK), jnp.int32),
             pltpu.VMEM((HALF, CHUNK), jnp.int32)]
            + [pltpu.VMEM((CHUNK, D), jnp.float32) for _ in range(NBUF)]
            + [pltpu.VMEM_SHARED((R, D), jnp.float32)]
            + [pltpu.SemaphoreType.DMA for _ in range(NBUF)]
        ),
    )
    def prop_kernel(xs_hbm, srcp_hbm, dstp_hbm, z_hbm, out_hbm,
                    sidx_v, didx_v, r0, r1, acc_sh, g0, g1):
        rows = (r0, r1)
        gsem = (g0, g1)
        c = lax.axis_index("c")
        s = lax.axis_index("s")
        w = c * NS + s
        pltpu.sync_copy(z_hbm, acc_sh.at[pl.ds(s * RPW, RPW)])
        plsc.subcore_barrier()

        for h in range(2):
            pltpu.sync_copy(srcp_hbm.at[w].at[h], sidx_v)
            pltpu.sync_copy(dstp_hbm.at[w].at[h], didx_v)

            @pl.loop(0, HALF)
            def _(g):
                pltpu.sync_copy(xs_hbm.at[sidx_v.at[g]], rows[0])
                pltpu.sync_copy(rows[0], acc_sh.at[didx_v.at[g]], add=True)

        plsc.subcore_barrier()
        pltpu.sync_copy(acc_sh.at[pl.ds(s * RPW, RPW)],
                        out_hbm.at[c].at[pl.ds(s * RPW, RPW)])

    return prop_kernel(xs, srcp, dstp, zrows)


BLK = 2000


def _perturb_body(degp_ref, p_ref, v_ref, dinv_ref, xs_ref):
    deg = degp_ref[0, :, 0:1] + degp_ref[1, :, 0:1]
    dv = lax.rsqrt(deg)
    dinv_ref[...] = dv
    sig = 1.0 / (1.0 + jnp.exp(-p_ref[...]))
    xs_ref[...] = (sig + v_ref[...]) * dv


@jax.jit
def _tc_perturb(degp, P_x, V_x):
    return pl.pallas_call(
        _perturb_body,
        grid=(N // BLK,),
        in_specs=[
            pl.BlockSpec((NC, BLK, 16), lambda i: (0, i, 0)),
            pl.BlockSpec((BLK, D), lambda i: (i, 0)),
            pl.BlockSpec((BLK, D), lambda i: (i, 0)),
        ],
        out_specs=[
            pl.BlockSpec((BLK, 1), lambda i: (i, 0)),
            pl.BlockSpec((BLK, D), lambda i: (i, 0)),
        ],
        out_shape=[
            jax.ShapeDtypeStruct((N, 1), jnp.float32),
            jax.ShapeDtypeStruct((N, D), jnp.float32),
        ],
    )(degp, P_x, V_x)


def _layer1_body(parts_ref, dinv_ref, w1_ref, hs_ref):
    dv = dinv_ref[...]
    x = (parts_ref[0] + parts_ref[1]) * dv
    h = jnp.maximum(jnp.dot(x, w1_ref[...], preferred_element_type=jnp.float32),
                    0.0)
    hs_ref[...] = h * dv


@jax.jit
def _tc_layer1(parts, dinv, W1):
    return pl.pallas_call(
        _layer1_body,
        grid=(N // BLK,),
        in_specs=[
            pl.BlockSpec((NC, BLK, D), lambda i: (0, i, 0)),
            pl.BlockSpec((BLK, 1), lambda i: (i, 0)),
            pl.BlockSpec((D, D), lambda i: (0, 0)),
        ],
        out_specs=pl.BlockSpec((BLK, D), lambda i: (i, 0)),
        out_shape=jax.ShapeDtypeStruct((N, D), jnp.float32),
    )(parts, dinv, W1)


def _layer2_body(parts_ref, dinv_ref, w2_ref, out_ref):
    dv = dinv_ref[...]
    x = (parts_ref[0] + parts_ref[1]) * dv
    logits = jnp.dot(x, w2_ref[...], preferred_element_type=jnp.float32)
    m = jnp.max(logits, axis=-1, keepdims=True)
    e = jnp.exp(logits - m)
    out_ref[...] = e / jnp.sum(e, axis=-1, keepdims=True)


@jax.jit
def _tc_layer2(parts, dinv, W2):
    return pl.pallas_call(
        _layer2_body,
        grid=(N // BLK,),
        in_specs=[
            pl.BlockSpec((NC, BLK, D), lambda i: (0, i, 0)),
            pl.BlockSpec((BLK, 1), lambda i: (i, 0)),
            pl.BlockSpec((D, C), lambda i: (0, 0)),
        ],
        out_specs=pl.BlockSpec((BLK, C), lambda i: (i, 0)),
        out_shape=jax.ShapeDtypeStruct((N, C), jnp.float32),
    )(parts, dinv, W2)


def kernel(V_x, adj, P_x, W1, W2):
    src = adj[0]
    dst = adj[1]
    loops = jnp.arange(N, dtype=jnp.int32)
    pad_s = jnp.zeros((E_PAD - E_TOT,), jnp.int32)
    pad_d = jnp.full((E_PAD - E_TOT,), N, jnp.int32)
    srcp = jnp.concatenate([src, loops, pad_s]).reshape(
        NC * NS, 2, HALF, CHUNK)
    srcp = jnp.pad(srcp, ((0, 0), (0, 0), (0, NBUF), (0, 0)))
    dstp = jnp.concatenate([dst, loops, pad_d]).reshape(
        NC * NS, 2, HALF, CHUNK)

    ones_rows = jnp.ones((CHUNK, 16), jnp.float32)
    z16 = jnp.zeros((RPW, 16), jnp.float32)
    zD = jnp.zeros((RPW, D), jnp.float32)

    degp = _sc_degree(dstp, ones_rows, z16)
    dinv, xs = _tc_perturb(degp, P_x, V_x)
    parts1 = _sc_propagate(xs, srcp, dstp, zD)
    hs = _tc_layer1(parts1, dinv, W1)
    parts2 = _sc_propagate(hs, srcp, dstp, zD)
    return _tc_layer2(parts2, dinv, W2)


# speedup vs baseline: 1.0030x; 1.0030x over previous
import functools

import jax
import jax.numpy as jnp
from jax import lax
from jax.experimental import pallas as pl
from jax.experimental.pallas import tpu as pltpu
from jax.experimental.pallas import tpu_sc as plsc

N = 10000
E = 320000
D = 128
C = 16

NC = 2
NS = 16
CHUNK = 128

E_TOT = E + N
NBUF = 2
NCHUNKS = 84
HALF = NCHUNKS // 2
EPW = NCHUNKS * CHUNK
E_PAD = EPW * NC * NS
R = 10112
RPW = R // NS

_mesh = plsc.VectorSubcoreMesh(core_axis_name="c", subcore_axis_name="s")


@jax.jit
def _sc_degree(dstp, ones_rows, zrows):
    @functools.partial(
        pl.kernel,
        mesh=_mesh,
        out_type=jax.ShapeDtypeStruct((NC, R, 16), jnp.float32),
        scratch_types=[
            pltpu.VMEM((2, HALF, CHUNK), jnp.int32),
            pltpu.VMEM((CHUNK, 16), jnp.float32),
            pltpu.VMEM_SHARED((R, 16), jnp.float32),
            pltpu.SemaphoreType.DMA,
        ],
    )
    def deg_kernel(dstp_hbm, ones_hbm, z_hbm, out_hbm, didx_v, ones_v, acc_sh,
                   sem):
        c = lax.axis_index("c")
        s = lax.axis_index("s")
        w = c * NS + s
        pltpu.sync_copy(dstp_hbm.at[w], didx_v)
        pltpu.sync_copy(ones_hbm, ones_v)
        pltpu.sync_copy(z_hbm, acc_sh.at[pl.ds(s * RPW, RPW)])
        plsc.subcore_barrier()

        for h in range(2):
            @pl.loop(0, HALF)
            def _(j):
                pltpu.sync_copy(ones_v, acc_sh.at[didx_v.at[h].at[j]],
                                add=True)

        plsc.subcore_barrier()
        pltpu.sync_copy(acc_sh.at[pl.ds(s * RPW, RPW)],
                        out_hbm.at[c].at[pl.ds(s * RPW, RPW)])

    return deg_kernel(dstp, ones_rows, zrows)


@jax.jit
def _sc_propagate(xs, srcp, dstp, zrows):
    @functools.partial(
        pl.kernel,
        mesh=_mesh,
        out_type=jax.ShapeDtypeStruct((NC, R, D), jnp.float32),
        scratch_types=(
            [pltpu.VMEM((HALF + NBUF, CHUNK), jnp.int32),
             pltpu.VMEM((HALF, CHUNK), jnp.int32)]
            + [pltpu.VMEM((CHUNK, D), jnp.float32) for _ in range(NBUF)]
            + [pltpu.VMEM_SHARED((R, D), jnp.float32)]
            + [pltpu.SemaphoreType.DMA for _ in range(NBUF)]
        ),
    )
    def prop_kernel(xs_hbm, srcp_hbm, dstp_hbm, z_hbm, out_hbm,
                    sidx_v, didx_v, r0, r1, acc_sh, g0, g1):
        rows = (r0, r1)
        gsem = (g0, g1)
        c = lax.axis_index("c")
        s = lax.axis_index("s")
        w = c * NS + s
        pltpu.sync_copy(z_hbm, acc_sh.at[pl.ds(s * RPW, RPW)])
        plsc.subcore_barrier()

        for h in range(2):
            pltpu.sync_copy(srcp_hbm.at[w].at[h], sidx_v)
            pltpu.sync_copy(dstp_hbm.at[w].at[h], didx_v)

            @pl.loop(0, HALF)
            def _(g):
                pltpu.sync_copy(xs_hbm.at[sidx_v.at[g]], rows[0])
                pltpu.sync_copy(rows[0], acc_sh.at[didx_v.at[g]], add=True)

        plsc.subcore_barrier()
        pltpu.sync_copy(acc_sh.at[pl.ds(s * RPW, RPW)],
                        out_hbm.at[c].at[pl.ds(s * RPW, RPW)])

    return prop_kernel(xs, srcp, dstp, zrows)


BLK = 2000


def _perturb_body(degp_ref, p_ref, v_ref, dinv_ref, xs_ref):
    deg = degp_ref[0, :, 0:1] + degp_ref[1, :, 0:1]
    dv = lax.rsqrt(deg)
    dinv_ref[...] = dv
    sig = 1.0 / (1.0 + jnp.exp(-p_ref[...]))
    xs_ref[...] = (sig + v_ref[...]) * dv


@jax.jit
def _tc_perturb(degp, P_x, V_x):
    return pl.pallas_call(
        _perturb_body,
        grid=(N // BLK,),
        in_specs=[
            pl.BlockSpec((NC, BLK, 16), lambda i: (0, i, 0)),
            pl.BlockSpec((BLK, D), lambda i: (i, 0)),
            pl.BlockSpec((BLK, D), lambda i: (i, 0)),
        ],
        out_specs=[
            pl.BlockSpec((BLK, 1), lambda i: (i, 0)),
            pl.BlockSpec((BLK, D), lambda i: (i, 0)),
        ],
        out_shape=[
            jax.ShapeDtypeStruct((N, 1), jnp.float32),
            jax.ShapeDtypeStruct((N, D), jnp.float32),
        ],
    )(degp, P_x, V_x)


def _layer1_body(parts_ref, dinv_ref, w1_ref, hs_ref):
    dv = dinv_ref[...]
    x = (parts_ref[0] + parts_ref[1]) * dv
    h = jnp.maximum(jnp.dot(x, w1_ref[...], preferred_element_type=jnp.float32),
                    0.0)
    hs_ref[...] = h * dv


@jax.jit
def _tc_layer1(parts, dinv, W1):
    return pl.pallas_call(
        _layer1_body,
        grid=(N // BLK,),
        in_specs=[
            pl.BlockSpec((NC, BLK, D), lambda i: (0, i, 0)),
            pl.BlockSpec((BLK, 1), lambda i: (i, 0)),
            pl.BlockSpec((D, D), lambda i: (0, 0)),
        ],
        out_specs=pl.BlockSpec((BLK, D), lambda i: (i, 0)),
        out_shape=jax.ShapeDtypeStruct((N, D), jnp.float32),
    )(parts, dinv, W1)


def _layer2_body(parts_ref, dinv_ref, w2_ref, out_ref):
    dv = dinv_ref[...]
    x = (parts_ref[0] + parts_ref[1]) * dv
    logits = jnp.dot(x, w2_ref[...], preferred_element_type=jnp.float32)
    m = jnp.max(logits, axis=-1, keepdims=True)
    e = jnp.exp(logits - m)
    out_ref[...] = e / jnp.sum(e, axis=-1, keepdims=True)


@jax.jit
def _tc_layer2(parts, dinv, W2):
    return pl.pallas_call(
        _layer2_body,
        grid=(N // BLK,),
        in_specs=[
            pl.BlockSpec((NC, BLK, D), lambda i: (0, i, 0)),
            pl.BlockSpec((BLK, 1), lambda i: (i, 0)),
            pl.BlockSpec((D, C), lambda i: (0, 0)),
        ],
        out_specs=pl.BlockSpec((BLK, C), lambda i: (i, 0)),
        out_shape=jax.ShapeDtypeStruct((N, C), jnp.float32),
    )(parts, dinv, W2)


def kernel(V_x, adj, P_x, W1, W2):
    src = adj[0]
    dst = adj[1]
    loops = jnp.arange(N, dtype=jnp.int32)
    pad_s = jnp.zeros((E_PAD - E_TOT,), jnp.int32)
    pad_d = jnp.full((E_PAD - E_TOT,), N, jnp.int32)
    srcp = jnp.concatenate([src, loops, pad_s]).reshape(
        NC * NS, 2, HALF, CHUNK)
    srcp = jnp.pad(srcp, ((0, 0), (0, 0), (0, NBUF), (0, 0)))
    dstp = jnp.concatenate([dst, loops, pad_d]).reshape(
        NC * NS, 2, HALF, CHUNK)

    ones_rows = jnp.ones((CHUNK, 16), jnp.float32)
    z16 = jnp.zeros((RPW, 16), jnp.float32)
    zD = jnp.zeros((RPW, D), jnp.float32)

    degp = _sc_degree(dstp, ones_rows, z16)
    dinv, xs = _tc_perturb(degp, P_x, V_x)
    parts1 = _sc_propagate(xs, srcp, dstp, zD)
    hs = _tc_layer1(parts1, dinv, W1)
    parts2 = _sc_propagate(hs, srcp, dstp, zD)
    return _tc_layer2(parts2, dinv, W2)


# speedup vs baseline: 2.4252x; 2.4179x over previous
import functools

import jax
import jax.numpy as jnp
from jax import lax
from jax.experimental import pallas as pl
from jax.experimental.pallas import tpu as pltpu
from jax.experimental.pallas import tpu_sc as plsc

N = 10000
E = 320000
D = 128
C = 16

NC = 2
NS = 16
CHUNK = 128

E_TOT = E + N
NCHUNKS = -(-E_TOT // (NC * NS * CHUNK))
EPW = NCHUNKS * CHUNK
E_PAD = EPW * NC * NS
R = 10112
RPW = R // NS

_mesh = plsc.VectorSubcoreMesh(core_axis_name="c", subcore_axis_name="s")


@jax.jit
def _sc_degree(dstp, ones_rows, zrows):
    @functools.partial(
        pl.kernel,
        mesh=_mesh,
        out_type=jax.ShapeDtypeStruct((NC, R, 16), jnp.float32),
        scratch_types=[
            pltpu.VMEM((NCHUNKS, CHUNK), jnp.int32),
            pltpu.VMEM((CHUNK, 16), jnp.float32),
            pltpu.VMEM_SHARED((R, 16), jnp.float32),
        ],
    )
    def deg_kernel(dstp_hbm, ones_hbm, z_hbm, out_hbm, didx_v, ones_v, acc_sh):
        c = lax.axis_index("c")
        s = lax.axis_index("s")
        w = c * NS + s
        pltpu.sync_copy(dstp_hbm.at[w], didx_v)
        pltpu.sync_copy(ones_hbm, ones_v)
        pltpu.sync_copy(z_hbm, acc_sh.at[pl.ds(s * RPW, RPW)])
        plsc.subcore_barrier()

        @pl.loop(0, NCHUNKS)
        def _(g):
            pltpu.sync_copy(ones_v, acc_sh.at[didx_v.at[g]], add=True)

        plsc.subcore_barrier()
        pltpu.sync_copy(acc_sh.at[pl.ds(s * RPW, RPW)],
                        out_hbm.at[c].at[pl.ds(s * RPW, RPW)])

    return deg_kernel(dstp, ones_rows, zrows)


@jax.jit
def _sc_propagate(xs, srcp, dstp, zrows):
    @functools.partial(
        pl.kernel,
        mesh=_mesh,
        out_type=jax.ShapeDtypeStruct((NC, R, D), jnp.float32),
        scratch_types=[
            pltpu.VMEM((NCHUNKS, CHUNK), jnp.int32),
            pltpu.VMEM((NCHUNKS, CHUNK), jnp.int32),
            pltpu.VMEM((CHUNK, D), jnp.float32),
            pltpu.VMEM_SHARED((R, D), jnp.float32),
        ],
    )
    def prop_kernel(xs_hbm, srcp_hbm, dstp_hbm, z_hbm, out_hbm,
                    sidx_v, didx_v, rows_v, acc_sh):
        c = lax.axis_index("c")
        s = lax.axis_index("s")
        w = c * NS + s
        pltpu.sync_copy(srcp_hbm.at[w], sidx_v)
        pltpu.sync_copy(dstp_hbm.at[w], didx_v)
        pltpu.sync_copy(z_hbm, acc_sh.at[pl.ds(s * RPW, RPW)])
        plsc.subcore_barrier()

        @pl.loop(0, NCHUNKS)
        def _(g):
            pltpu.sync_copy(xs_hbm.at[sidx_v.at[g]], rows_v)
            pltpu.sync_copy(rows_v, acc_sh.at[didx_v.at[g]], add=True)

        plsc.subcore_barrier()
        pltpu.sync_copy(acc_sh.at[pl.ds(s * RPW, RPW)],
                        out_hbm.at[c].at[pl.ds(s * RPW, RPW)])

    return prop_kernel(xs, srcp, dstp, zrows)


BLK = 2000


def _perturb_body(degp_ref, p_ref, v_ref, dinv_ref, xs_ref):
    deg = degp_ref[0, :, 0:1] + degp_ref[1, :, 0:1]
    dv = lax.rsqrt(deg)
    dinv_ref[...] = dv
    sig = 1.0 / (1.0 + jnp.exp(-p_ref[...]))
    xs_ref[...] = (sig + v_ref[...]) * dv


@jax.jit
def _tc_perturb(degp, P_x, V_x):
    return pl.pallas_call(
        _perturb_body,
        grid=(N // BLK,),
        in_specs=[
            pl.BlockSpec((NC, BLK, 16), lambda i: (0, i, 0)),
            pl.BlockSpec((BLK, D), lambda i: (i, 0)),
            pl.BlockSpec((BLK, D), lambda i: (i, 0)),
        ],
        out_specs=[
            pl.BlockSpec((BLK, 1), lambda i: (i, 0)),
            pl.BlockSpec((BLK, D), lambda i: (i, 0)),
        ],
        out_shape=[
            jax.ShapeDtypeStruct((N, 1), jnp.float32),
            jax.ShapeDtypeStruct((N, D), jnp.float32),
        ],
    )(degp, P_x, V_x)


def _layer1_body(parts_ref, dinv_ref, w1_ref, hs_ref):
    dv = dinv_ref[...]
    x = (parts_ref[0] + parts_ref[1]) * dv
    h = jnp.maximum(jnp.dot(x, w1_ref[...], preferred_element_type=jnp.float32),
                    0.0)
    hs_ref[...] = h * dv


@jax.jit
def _tc_layer1(parts, dinv, W1):
    return pl.pallas_call(
        _layer1_body,
        grid=(N // BLK,),
        in_specs=[
            pl.BlockSpec((NC, BLK, D), lambda i: (0, i, 0)),
            pl.BlockSpec((BLK, 1), lambda i: (i, 0)),
            pl.BlockSpec((D, D), lambda i: (0, 0)),
        ],
        out_specs=pl.BlockSpec((BLK, D), lambda i: (i, 0)),
        out_shape=jax.ShapeDtypeStruct((N, D), jnp.float32),
    )(parts, dinv, W1)


def _layer2_body(parts_ref, dinv_ref, w2_ref, out_ref):
    dv = dinv_ref[...]
    x = (parts_ref[0] + parts_ref[1]) * dv
    logits = jnp.dot(x, w2_ref[...], preferred_element_type=jnp.float32)
    m = jnp.max(logits, axis=-1, keepdims=True)
    e = jnp.exp(logits - m)
    out_ref[...] = e / jnp.sum(e, axis=-1, keepdims=True)


@jax.jit
def _tc_layer2(parts, dinv, W2):
    return pl.pallas_call(
        _layer2_body,
        grid=(N // BLK,),
        in_specs=[
            pl.BlockSpec((NC, BLK, D), lambda i: (0, i, 0)),
            pl.BlockSpec((BLK, 1), lambda i: (i, 0)),
            pl.BlockSpec((D, C), lambda i: (0, 0)),
        ],
        out_specs=pl.BlockSpec((BLK, C), lambda i: (i, 0)),
        out_shape=jax.ShapeDtypeStruct((N, C), jnp.float32),
    )(parts, dinv, W2)


def kernel(V_x, adj, P_x, W1, W2):
    src = adj[0]
    dst = adj[1]
    loops = jnp.arange(N, dtype=jnp.int32)
    pad_s = jnp.zeros((E_PAD - E_TOT,), jnp.int32)
    pad_d = jnp.full((E_PAD - E_TOT,), N, jnp.int32)
    srcp = jnp.concatenate([src, loops, pad_s]).reshape(NC * NS, NCHUNKS, CHUNK)
    dstp = jnp.concatenate([dst, loops, pad_d]).reshape(NC * NS, NCHUNKS, CHUNK)

    ones_rows = jnp.ones((CHUNK, 16), jnp.float32)
    z16 = jnp.zeros((RPW, 16), jnp.float32)
    zD = jnp.zeros((RPW, D), jnp.float32)

    degp = _sc_degree(dstp, ones_rows, z16)
    dinv, xs = _tc_perturb(degp, P_x, V_x)
    parts1 = _sc_propagate(xs, srcp, dstp, zD)
    hs = _tc_layer1(parts1, dinv, W1)
    parts2 = _sc_propagate(hs, srcp, dstp, zD)
    return _tc_layer2(parts2, dinv, W2)
